# baseline (speedup 1.0000x reference)
import functools

import jax
import jax.numpy as jnp
from jax import lax
from jax.experimental import pallas as pl
from jax.experimental.pallas import tpu as pltpu
from jax.experimental.pallas import tpu_sc as plsc

N = 10000
NPAD = 10240
E = 320000
D = 128
G = 512
NCLS = 2
NC = 2
NS = 16
NW = NC * NS
K = 128
ECH = 2560
EPAD = ECH * K
CPT = ECH // NW
EPT = E // NW
RPT = NPAD // NS

_mesh = plsc.VectorSubcoreMesh(core_axis_name="c", subcore_axis_name="s")


_EB = 2048
_NHB = EPAD // _EB


def _hist_body(dst_ref, hist_ref):
    d = dst_ref[...]
    lanes = lax.broadcasted_iota(jnp.int32, (1, D), 1)
    a = ((d >> 7) == lanes).astype(jnp.float32)
    b = ((d & 127) == lanes).astype(jnp.float32)
    dn = (((0,), (0,)), ((), ()))
    contrib = lax.dot_general(a, b, dn, preferred_element_type=jnp.float32)

    @pl.when(pl.program_id(0) == 0)
    def _init():
        hist_ref[...] = jnp.zeros((D, D), jnp.float32)

    hist_ref[...] += contrib


_hist_tc = pl.pallas_call(
    _hist_body,
    grid=(_NHB,),
    in_specs=[pl.BlockSpec((_EB, 1), lambda i: (i, 0))],
    out_specs=pl.BlockSpec((D, D), lambda i: (0, 0)),
    out_shape=jax.ShapeDtypeStruct((D, D), jnp.float32),
)


IB = 16


@functools.partial(
    pl.kernel,
    out_type=jax.ShapeDtypeStruct((NC, NPAD, D), jnp.float32),
    mesh=_mesh,
    scratch_types=[
        pltpu.VMEM((IB, K), jnp.int32),
        pltpu.VMEM((IB, K), jnp.int32),
        pltpu.VMEM((K, D), jnp.float32),
        pltpu.VMEM((K, D), jnp.float32),
        pltpu.VMEM_SHARED((NPAD, D), jnp.float32),
        pltpu.SemaphoreType.DMA,
        pltpu.SemaphoreType.DMA,
    ],
)
def _edge_agg(g_hbm, src_hbm, dst_hbm, out_hbm, src_blk, dst_blk, rows_a,
              rows_b, acc_sh, sem_a, sem_b):
    c = lax.axis_index("c")
    s = lax.axis_index("s")
    wid = s * NC + c
    zeros16 = jnp.zeros((16,), jnp.float32)

    def zrow(i, carry):
        for cc in range(D // 16):
            rows_a[i, pl.ds(cc * 16, 16)] = zeros16
        return carry

    lax.fori_loop(0, K, zrow, 0)
    for r in range(RPT // K):
        pltpu.sync_copy(rows_a, acc_sh.at[pl.ds(s * RPT + r * K, K)])
    plsc.subcore_barrier()

    def inner(j, carry):
        t0 = 2 * j
        pltpu.sync_copy(rows_a, acc_sh.at[dst_blk.at[t0]], add=True)
        na = jnp.minimum(t0 + 2, IB - 2)
        ga = pltpu.async_copy(g_hbm.at[src_blk.at[na]], rows_a, sem_a)
        pltpu.sync_copy(rows_b, acc_sh.at[dst_blk.at[t0 + 1]], add=True)
        nb = jnp.minimum(t0 + 3, IB - 1)
        gb = pltpu.async_copy(g_hbm.at[src_blk.at[nb]], rows_b, sem_b)
        ga.wait()
        gb.wait()
        return carry

    def outer(b, carry):
        base = wid * CPT + b * IB
        pltpu.sync_copy(src_hbm.at[pl.ds(base, IB)], src_blk)
        pltpu.sync_copy(dst_hbm.at[pl.ds(base, IB)], dst_blk)
        pltpu.async_copy(g_hbm.at[src_blk.at[0]], rows_a, sem_a).wait()
        pltpu.async_copy(g_hbm.at[src_blk.at[1]], rows_b, sem_b).wait()
        lax.fori_loop(0, IB // 2, inner, 0)
        return carry

    lax.fori_loop(0, CPT // IB, outer, 0)
    plsc.subcore_barrier()
    pltpu.sync_copy(acc_sh.at[pl.ds(s * RPT, RPT)],
                    out_hbm.at[c, pl.ds(s * RPT, RPT)])


def _scale_body(x_ref, wt_ref, deg_ref, g_ref, dinv_ref):
    deg = deg_ref[...] + 1.0
    dinv = lax.rsqrt(deg)
    h = jnp.dot(x_ref[...], wt_ref[...], preferred_element_type=jnp.float32)
    g_ref[...] = h * dinv
    dinv_ref[...] = dinv


BN_ROWS = 1000

_pass2 = pl.pallas_call(
    _scale_body,
    grid=(N // BN_ROWS,),
    in_specs=[
        pl.BlockSpec((BN_ROWS, D), lambda i: (i, 0)),
        pl.BlockSpec((D, D), lambda i: (0, 0)),
        pl.BlockSpec((BN_ROWS, 1), lambda i: (i, 0)),
    ],
    out_specs=[
        pl.BlockSpec((BN_ROWS, D), lambda i: (i, 0)),
        pl.BlockSpec((BN_ROWS, 1), lambda i: (i, 0)),
    ],
    out_shape=[
        jax.ShapeDtypeStruct((N, D), jnp.float32),
        jax.ShapeDtypeStruct((N, 1), jnp.float32),
    ],
)


_CH = 1000


def _head_body(agg_ref, g_ref, dinv_ref, batch_ref, bconv_ref, bn1w_ref,
               bn1b_ref, linw_ref, linb_ref, bn2w_ref, bn2b_ref, out_ref):
    eps = 1e-5
    agg = agg_ref[0, :N, :] + agg_ref[1, :N, :]
    conv = (agg + g_ref[...]) * dinv_ref[...] + bconv_ref[...]
    m1 = jnp.mean(conv, axis=0, keepdims=True)
    v1 = jnp.mean((conv - m1) ** 2, axis=0, keepdims=True)
    h = jnp.maximum(
        (conv - m1) * lax.rsqrt(v1 + eps) * bn1w_ref[...] + bn1b_ref[...], 0.0)
    iota_g = lax.broadcasted_iota(jnp.int32, (1, G), 1)
    ones_chunk = jnp.ones((_CH, D), jnp.float32)
    acc = jnp.zeros((G, D), jnp.float32)
    cnt = jnp.zeros((G, D), jnp.float32)
    dn = (((0,), (0,)), ((), ()))
    for r in range(N // _CH):
        a = (batch_ref[r * _CH:(r + 1) * _CH, :] == iota_g).astype(jnp.float32)
        hc = h[r * _CH:(r + 1) * _CH, :]
        acc = acc + lax.dot_general(a, hc, dn,
                                    preferred_element_type=jnp.float32)
        cnt = cnt + lax.dot_general(a, ones_chunk, dn,
                                    preferred_element_type=jnp.float32)
    pooled = acc / jnp.maximum(cnt, 1.0)
    o = jnp.dot(pooled, linw_ref[...],
                preferred_element_type=jnp.float32) + linb_ref[...]
    m2 = jnp.mean(o, axis=0, keepdims=True)
    v2 = jnp.mean((o - m2) ** 2, axis=0, keepdims=True)
    out_ref[...] = (o - m2) * lax.rsqrt(v2 + eps) * bn2w_ref[...] + bn2b_ref[...]


_pass4 = pl.pallas_call(
    _head_body,
    out_shape=jax.ShapeDtypeStruct((G, D), jnp.float32),
)


def kernel(x, edge_index, batch, W_conv, b_conv, bn1_w, bn1_b, lin_w, lin_b,
           bn2_w, bn2_b):
    src = edge_index[0].astype(jnp.int32)
    dst = edge_index[1].astype(jnp.int32)
    pad = EPAD - E
    src_p = jnp.concatenate([src, jnp.zeros((pad,), jnp.int32)]).reshape(ECH, K)
    dst_p = jnp.concatenate([dst, jnp.full((pad,), N, jnp.int32)]).reshape(ECH, K)
    hist = _hist_tc(dst_p.reshape(EPAD, 1))
    deg = hist.reshape(D * D)[:N][:, None]
    g, dinv = _pass2(x, W_conv.T, deg)
    agg = _edge_agg(g, src_p, dst_p)
    batch2d = batch.astype(jnp.int32)[:, None]
    linw_pad = jnp.zeros((D, D), jnp.float32).at[:, :NCLS].set(lin_w.T)
    linb_pad = jnp.zeros((1, D), jnp.float32).at[0, :NCLS].set(lin_b)
    bn2w_pad = jnp.zeros((1, D), jnp.float32).at[0, :NCLS].set(bn2_w)
    bn2b_pad = jnp.zeros((1, D), jnp.float32).at[0, :NCLS].set(bn2_b)
    out = _pass4(agg, g, dinv, batch2d, b_conv[None, :], bn1_w[None, :],
                 bn1_b[None, :], linw_pad, linb_pad, bn2w_pad, bn2b_pad)
    return out[:, :NCLS]


# speedup vs baseline: 1.9633x; 1.9633x over previous
import functools

import jax
import jax.numpy as jnp
from jax import lax
from jax.experimental import pallas as pl
from jax.experimental.pallas import tpu as pltpu
from jax.experimental.pallas import tpu_sc as plsc

N = 10000
NPAD = 10240
E = 320000
D = 128
G = 512
NCLS = 2
NC = 2
NS = 16
NW = NC * NS
K = 125
ECH = E // K
CPT = ECH // NW
RPT = NPAD // NS

_mesh = plsc.VectorSubcoreMesh(core_axis_name="c", subcore_axis_name="s")


_EB = 2000
_NHB = E // _EB


def _hist_body(dst_ref, hist_ref):
    d = dst_ref[...]
    lanes = lax.broadcasted_iota(jnp.int32, (1, D), 1)
    a = ((d >> 7) == lanes).astype(jnp.float32)
    b = ((d & 127) == lanes).astype(jnp.float32)
    dn = (((0,), (0,)), ((), ()))
    contrib = lax.dot_general(a, b, dn, preferred_element_type=jnp.float32)

    @pl.when(pl.program_id(0) == 0)
    def _init():
        hist_ref[...] = jnp.zeros((D, D), jnp.float32)

    hist_ref[...] += contrib


_hist_tc = pl.pallas_call(
    _hist_body,
    grid=(_NHB,),
    in_specs=[pl.BlockSpec((_EB, 1), lambda i: (i, 0))],
    out_specs=pl.BlockSpec((D, D), lambda i: (0, 0)),
    out_shape=jax.ShapeDtypeStruct((D, D), jnp.float32),
)


HC = CPT // 2


@functools.partial(
    pl.kernel,
    out_type=jax.ShapeDtypeStruct((NC, NPAD, D), jnp.float32),
    mesh=_mesh,
    scratch_types=[
        pltpu.VMEM((HC, K), jnp.int32),
        pltpu.VMEM((HC, K), jnp.int32),
        pltpu.VMEM((128, D), jnp.float32),
        pltpu.VMEM((128, D), jnp.float32),
        pltpu.VMEM_SHARED((NPAD, D), jnp.float32),
        pltpu.SemaphoreType.DMA,
        pltpu.SemaphoreType.DMA,
        pltpu.SemaphoreType.DMA,
        pltpu.SemaphoreType.DMA,
    ],
)
def _edge_agg(g_hbm, src_hbm, dst_hbm, out_hbm, src_blk, dst_blk, rows0,
              rows1, acc_sh, sg0, sg1, ss0, ss1):
    c = lax.axis_index("c")
    s = lax.axis_index("s")
    wid = s * NC + c
    zeros16 = jnp.zeros((16,), jnp.float32)

    def zrow(i, carry):
        for cc in range(D // 16):
            rows0[i, pl.ds(cc * 16, 16)] = zeros16
        return carry

    lax.fori_loop(0, 128, zrow, 0)
    for r in range(RPT // 128):
        pltpu.sync_copy(rows0, acc_sh.at[pl.ds(s * RPT + r * 128, 128)])
    plsc.subcore_barrier()

    r0 = rows0.at[pl.ds(0, K)]
    r1 = rows1.at[pl.ds(0, K)]

    def inner(i, carry):
        t0 = 2 * i
        t1 = t0 + 1
        pltpu.make_async_copy(g_hbm.at[src_blk.at[t0]], r0, sg0).wait()
        sc0 = pltpu.make_async_copy(r0, acc_sh.at[dst_blk.at[t0]], ss0)
        sc0.start(add=True)
        pltpu.make_async_copy(g_hbm.at[src_blk.at[t1]], r1, sg1).wait()
        sc1 = pltpu.make_async_copy(r1, acc_sh.at[dst_blk.at[t1]], ss1)
        sc1.start(add=True)
        n0 = jnp.minimum(t0 + 2, HC - 2)
        n1 = jnp.minimum(t1 + 2, HC - 1)
        pltpu.make_async_copy(r0, acc_sh.at[dst_blk.at[t0]], ss0).wait()
        pltpu.make_async_copy(g_hbm.at[src_blk.at[n0]], r0, sg0).start()
        pltpu.make_async_copy(r1, acc_sh.at[dst_blk.at[t1]], ss1).wait()
        pltpu.make_async_copy(g_hbm.at[src_blk.at[n1]], r1, sg1).start()
        return carry

    for h in range(2):
        base = wid * CPT + h * HC
        pltpu.sync_copy(src_hbm.at[pl.ds(base, HC)], src_blk)
        pltpu.sync_copy(dst_hbm.at[pl.ds(base, HC)], dst_blk)
        pltpu.make_async_copy(g_hbm.at[src_blk.at[0]], r0, sg0).start()
        pltpu.make_async_copy(g_hbm.at[src_blk.at[1]], r1, sg1).start()
        lax.fori_loop(0, HC // 2, inner, 0)
        pltpu.make_async_copy(g_hbm.at[src_blk.at[0]], r0, sg0).wait()
        pltpu.make_async_copy(g_hbm.at[src_blk.at[1]], r1, sg1).wait()

    plsc.subcore_barrier()
    pltpu.sync_copy(acc_sh.at[pl.ds(s * RPT, RPT)],
                    out_hbm.at[c, pl.ds(s * RPT, RPT)])


def _scale_body(x_ref, wt_ref, deg_ref, g_ref, dinv_ref):
    deg = deg_ref[...] + 1.0
    dinv = lax.rsqrt(deg)
    h = jnp.dot(x_ref[...], wt_ref[...], preferred_element_type=jnp.float32)
    g_ref[...] = h * dinv
    dinv_ref[...] = dinv


BN_ROWS = 1000

_pass2 = pl.pallas_call(
    _scale_body,
    grid=(N // BN_ROWS,),
    in_specs=[
        pl.BlockSpec((BN_ROWS, D), lambda i: (i, 0)),
        pl.BlockSpec((D, D), lambda i: (0, 0)),
        pl.BlockSpec((BN_ROWS, 1), lambda i: (i, 0)),
    ],
    out_specs=[
        pl.BlockSpec((BN_ROWS, D), lambda i: (i, 0)),
        pl.BlockSpec((BN_ROWS, 1), lambda i: (i, 0)),
    ],
    out_shape=[
        jax.ShapeDtypeStruct((N, D), jnp.float32),
        jax.ShapeDtypeStruct((N, 1), jnp.float32),
    ],
)


_CH = 1000


def _head_body(agg_ref, g_ref, dinv_ref, batch_ref, bconv_ref, bn1w_ref,
               bn1b_ref, linw_ref, linb_ref, bn2w_ref, bn2b_ref, out_ref):
    eps = 1e-5
    agg = agg_ref[0, :N, :] + agg_ref[1, :N, :]
    conv = (agg + g_ref[...]) * dinv_ref[...] + bconv_ref[...]
    m1 = jnp.mean(conv, axis=0, keepdims=True)
    v1 = jnp.mean((conv - m1) ** 2, axis=0, keepdims=True)
    h = jnp.maximum(
        (conv - m1) * lax.rsqrt(v1 + eps) * bn1w_ref[...] + bn1b_ref[...], 0.0)
    iota_g = lax.broadcasted_iota(jnp.int32, (1, G), 1)
    ones_chunk = jnp.ones((_CH, D), jnp.float32)
    acc = jnp.zeros((G, D), jnp.float32)
    cnt = jnp.zeros((G, D), jnp.float32)
    dn = (((0,), (0,)), ((), ()))
    for r in range(N // _CH):
        a = (batch_ref[r * _CH:(r + 1) * _CH, :] == iota_g).astype(jnp.float32)
        hc = h[r * _CH:(r + 1) * _CH, :]
        acc = acc + lax.dot_general(a, hc, dn,
                                    preferred_element_type=jnp.float32)
        cnt = cnt + lax.dot_general(a, ones_chunk, dn,
                                    preferred_element_type=jnp.float32)
    pooled = acc / jnp.maximum(cnt, 1.0)
    dn_nt = (((1,), (1,)), ((), ()))
    o = lax.dot_general(pooled, linw_ref[...], dn_nt,
                        preferred_element_type=jnp.float32) + linb_ref[...]
    m2 = jnp.mean(o, axis=0, keepdims=True)
    v2 = jnp.mean((o - m2) ** 2, axis=0, keepdims=True)
    out_ref[...] = (o - m2) * lax.rsqrt(v2 + eps) * bn2w_ref[...] + bn2b_ref[...]


_pass4 = pl.pallas_call(
    _head_body,
    out_shape=jax.ShapeDtypeStruct((G, NCLS), jnp.float32),
)


def kernel(x, edge_index, batch, W_conv, b_conv, bn1_w, bn1_b, lin_w, lin_b,
           bn2_w, bn2_b):
    src = edge_index[0].astype(jnp.int32)
    dst = edge_index[1].astype(jnp.int32)
    src_p = src.reshape(ECH, K)
    dst_p = dst.reshape(ECH, K)
    hist = _hist_tc(dst.reshape(E, 1))
    deg = hist.reshape(D * D, 1)
    g, dinv = _pass2(x, W_conv.T, deg)
    agg = _edge_agg(g, src_p, dst_p)
    batch2d = batch.astype(jnp.int32)[:, None]
    return _pass4(agg, g, dinv, batch2d, b_conv[None, :], bn1_w[None, :],
                  bn1_b[None, :], lin_w, lin_b[None, :], bn2_w[None, :],
                  bn2_b[None, :])


# speedup vs baseline: 1.9905x; 1.0139x over previous
import functools

import jax
import jax.numpy as jnp
from jax import lax
from jax.experimental import pallas as pl
from jax.experimental.pallas import tpu as pltpu
from jax.experimental.pallas import tpu_sc as plsc

N = 10000
NPAD = 10240
E = 320000
D = 128
G = 512
NCLS = 2
NC = 2
NS = 16
NW = NC * NS
K = 125
ECH = E // K
CPT = ECH // NW
RPT = NPAD // NS

_mesh = plsc.VectorSubcoreMesh(core_axis_name="c", subcore_axis_name="s")


_EB = 2000
_NHB = E // _EB


def _hist_scale_body(dst_ref, x_ref, wt_ref, g_ref, dinv_ref, hist_ref):
    i = pl.program_id(0)
    lanes = lax.broadcasted_iota(jnp.int32, (1, D), 1)

    @pl.when(i == 0)
    def _init():
        hist_ref[...] = jnp.zeros((D, D), jnp.float32)

    @pl.when(i < _NHB)
    def _hist():
        d = dst_ref[...]
        a = ((d >> 7) == lanes).astype(jnp.float32)
        b = ((d & 127) == lanes).astype(jnp.float32)
        dn = (((0,), (0,)), ((), ()))
        hist_ref[...] += lax.dot_general(a, b, dn,
                                         preferred_element_type=jnp.float32)

    @pl.when(i == _NHB)
    def _scale():
        n = lax.broadcasted_iota(jnp.int32, (N, D), 0)
        oh_hi = ((n >> 7) == lanes).astype(jnp.float32)
        oh_lo = ((n & 127) == lanes).astype(jnp.float32)
        v = jnp.dot(oh_hi, hist_ref[...], preferred_element_type=jnp.float32)
        deg = jnp.dot(v * oh_lo, jnp.ones((D, D), jnp.float32),
                      preferred_element_type=jnp.float32)
        dinv = lax.rsqrt(deg + 1.0)
        h = jnp.dot(x_ref[...], wt_ref[...],
                    preferred_element_type=jnp.float32)
        g_ref[...] = h * dinv
        dinv_ref[...] = dinv


_hist_scale = pl.pallas_call(
    _hist_scale_body,
    grid=(_NHB + 1,),
    in_specs=[
        pl.BlockSpec((_EB, 1), lambda i: (jnp.minimum(i, _NHB - 1), 0)),
        pl.BlockSpec((N, D), lambda i: (0, 0)),
        pl.BlockSpec((D, D), lambda i: (0, 0)),
    ],
    out_specs=[
        pl.BlockSpec((N, D), lambda i: (0, 0)),
        pl.BlockSpec((N, D), lambda i: (0, 0)),
    ],
    out_shape=[
        jax.ShapeDtypeStruct((N, D), jnp.float32),
        jax.ShapeDtypeStruct((N, D), jnp.float32),
    ],
    scratch_shapes=[pltpu.VMEM((D, D), jnp.float32)],
)


HC = CPT // 2


@functools.partial(
    pl.kernel,
    out_type=jax.ShapeDtypeStruct((NC, NPAD, D), jnp.float32),
    mesh=_mesh,
    scratch_types=[
        pltpu.VMEM((HC, K), jnp.int32),
        pltpu.VMEM((HC, K), jnp.int32),
        pltpu.VMEM((128, D), jnp.float32),
        pltpu.VMEM((128, D), jnp.float32),
        pltpu.VMEM_SHARED((NPAD, D), jnp.float32),
        pltpu.SemaphoreType.DMA,
        pltpu.SemaphoreType.DMA,
        pltpu.SemaphoreType.DMA,
        pltpu.SemaphoreType.DMA,
    ],
)
def _edge_agg(g_hbm, src_hbm, dst_hbm, out_hbm, src_blk, dst_blk, rows0,
              rows1, acc_sh, sg0, sg1, ss0, ss1):
    c = lax.axis_index("c")
    s = lax.axis_index("s")
    wid = s * NC + c
    zeros16 = jnp.zeros((16,), jnp.float32)

    def zrow(i, carry):
        for cc in range(D // 16):
            rows0[i, pl.ds(cc * 16, 16)] = zeros16
        return carry

    lax.fori_loop(0, 128, zrow, 0)
    for r in range(RPT // 128):
        pltpu.sync_copy(rows0, acc_sh.at[pl.ds(s * RPT + r * 128, 128)])
    plsc.subcore_barrier()

    r0 = rows0.at[pl.ds(0, K)]
    r1 = rows1.at[pl.ds(0, K)]

    def inner(i, carry):
        t0 = 2 * i
        t1 = t0 + 1
        pltpu.make_async_copy(g_hbm.at[src_blk.at[t0]], r0, sg0).wait()
        sc0 = pltpu.make_async_copy(r0, acc_sh.at[dst_blk.at[t0]], ss0)
        sc0.start(add=True)
        pltpu.make_async_copy(g_hbm.at[src_blk.at[t1]], r1, sg1).wait()
        sc1 = pltpu.make_async_copy(r1, acc_sh.at[dst_blk.at[t1]], ss1)
        sc1.start(add=True)
        n0 = jnp.minimum(t0 + 2, HC - 2)
        n1 = jnp.minimum(t1 + 2, HC - 1)
        pltpu.make_async_copy(r0, acc_sh.at[dst_blk.at[t0]], ss0).wait()
        pltpu.make_async_copy(g_hbm.at[src_blk.at[n0]], r0, sg0).start()
        pltpu.make_async_copy(r1, acc_sh.at[dst_blk.at[t1]], ss1).wait()
        pltpu.make_async_copy(g_hbm.at[src_blk.at[n1]], r1, sg1).start()
        return carry

    for h in range(2):
        base = wid * CPT + h * HC
        pltpu.sync_copy(src_hbm.at[pl.ds(base, HC)], src_blk)
        pltpu.sync_copy(dst_hbm.at[pl.ds(base, HC)], dst_blk)
        pltpu.make_async_copy(g_hbm.at[src_blk.at[0]], r0, sg0).start()
        pltpu.make_async_copy(g_hbm.at[src_blk.at[1]], r1, sg1).start()
        lax.fori_loop(0, HC // 2, inner, 0)
        pltpu.make_async_copy(g_hbm.at[src_blk.at[0]], r0, sg0).wait()
        pltpu.make_async_copy(g_hbm.at[src_blk.at[1]], r1, sg1).wait()

    plsc.subcore_barrier()
    pltpu.sync_copy(acc_sh.at[pl.ds(s * RPT, RPT)],
                    out_hbm.at[c, pl.ds(s * RPT, RPT)])


_CH = 1000


def _head_body(agg_ref, g_ref, dinv_ref, batch_ref, bconv_ref, bn1w_ref,
               bn1b_ref, linw_ref, linb_ref, bn2w_ref, bn2b_ref, out_ref):
    eps = 1e-5
    agg = agg_ref[0, :N, :] + agg_ref[1, :N, :]
    conv = (agg + g_ref[...]) * dinv_ref[...] + bconv_ref[...]
    m1 = jnp.mean(conv, axis=0, keepdims=True)
    v1 = jnp.mean((conv - m1) ** 2, axis=0, keepdims=True)
    h = jnp.maximum(
        (conv - m1) * lax.rsqrt(v1 + eps) * bn1w_ref[...] + bn1b_ref[...], 0.0)
    iota_g = lax.broadcasted_iota(jnp.int32, (1, G), 1)
    ones_chunk = jnp.ones((_CH, D), jnp.float32)
    acc = jnp.zeros((G, D), jnp.float32)
    cnt = jnp.zeros((G, D), jnp.float32)
    dn = (((0,), (0,)), ((), ()))
    for r in range(N // _CH):
        a = (batch_ref[r * _CH:(r + 1) * _CH, :] == iota_g).astype(jnp.float32)
        hc = h[r * _CH:(r + 1) * _CH, :]
        acc = acc + lax.dot_general(a, hc, dn,
                                    preferred_element_type=jnp.float32)
        cnt = cnt + lax.dot_general(a, ones_chunk, dn,
                                    preferred_element_type=jnp.float32)
    pooled = acc / jnp.maximum(cnt, 1.0)
    dn_nt = (((1,), (1,)), ((), ()))
    o = lax.dot_general(pooled, linw_ref[...], dn_nt,
                        preferred_element_type=jnp.float32) + linb_ref[...]
    m2 = jnp.mean(o, axis=0, keepdims=True)
    v2 = jnp.mean((o - m2) ** 2, axis=0, keepdims=True)
    out_ref[...] = (o - m2) * lax.rsqrt(v2 + eps) * bn2w_ref[...] + bn2b_ref[...]


_pass4 = pl.pallas_call(
    _head_body,
    out_shape=jax.ShapeDtypeStruct((G, NCLS), jnp.float32),
)


def kernel(x, edge_index, batch, W_conv, b_conv, bn1_w, bn1_b, lin_w, lin_b,
           bn2_w, bn2_b):
    src = edge_index[0].astype(jnp.int32)
    dst = edge_index[1].astype(jnp.int32)
    src_p = src.reshape(ECH, K)
    dst_p = dst.reshape(ECH, K)
    g, dinv = _hist_scale(dst.reshape(E, 1), x, W_conv.T)
    agg = _edge_agg(g, src_p, dst_p)
    batch2d = batch.astype(jnp.int32)[:, None]
    return _pass4(agg, g, dinv, batch2d, b_conv[None, :], bn1_w[None, :],
                  bn1_b[None, :], lin_w, lin_b[None, :], bn2_w[None, :],
                  bn2_b[None, :])
